# Initial kernel scaffold; baseline (speedup 1.0000x reference)
import functools

import jax
import jax.numpy as jnp
from jax import lax
from jax.experimental import pallas as pl
from jax.experimental.pallas import tpu as pltpu
from jax.experimental.pallas import tpu_sc as plsc

N = 10000
C = 128
P = 320000
E = 5000
TAU = 1.0

NC = 2
NS = 16
NW = NC * NS
K = 80
PER_W = P // NW
CHUNKS = PER_W // K
W = 2 * C + 16
EP = 5120
ROWS_PER_TILE = EP // NS
OB = 32


def _table_body(s_ref, t_ref, o_ref):
    def softmax(x):
        m = jnp.max(x, axis=1, keepdims=True)
        ex = jnp.exp(x - m)
        return ex / jnp.sum(ex, axis=1, keepdims=True)

    s = softmax(s_ref[...])
    t = softmax(t_ref[...])
    r = s.shape[0]
    col = lax.broadcasted_iota(jnp.int32, (r, 16), 1)
    extra = jnp.where(col == 0, 1.0, 0.0).astype(jnp.float32)
    o_ref[...] = jnp.concatenate([s, t, extra], axis=1)


def _build_table(pred_s, pred_t):
    R = 400
    return pl.pallas_call(
        _table_body,
        grid=(N // R,),
        in_specs=[pl.BlockSpec((R, C), lambda i: (i, 0)),
                  pl.BlockSpec((R, C), lambda i: (i, 0))],
        out_specs=pl.BlockSpec((R, W), lambda i: (i, 0)),
        out_shape=jax.ShapeDtypeStruct((N, W), jnp.float32),
    )(pred_s, pred_t)


def _sc_body(table_hbm, vidx_hbm, eidx_hbm, out_hbm,
             idx_v, idx_e, rows_v, stage_v, acc_sh, sem):
    cid = lax.axis_index("c")
    sid = lax.axis_index("s")
    wid = sid * NC + cid

    zero = jnp.zeros((16,), jnp.float32)

    def zstore(i, carry):
        r = i // (W // 16)
        c = i % (W // 16)
        stage_v[r, pl.ds(c * 16, 16)] = zero
        return carry

    lax.fori_loop(0, OB * (W // 16), zstore, 0)

    def zcopy(j, carry):
        r0 = sid * ROWS_PER_TILE + j * OB
        pltpu.sync_copy(stage_v, acc_sh.at[pl.ds(r0, OB)])
        return carry

    lax.fori_loop(0, ROWS_PER_TILE // OB, zcopy, 0)
    plsc.subcore_barrier()

    base = wid * PER_W

    def chunk(i, carry):
        off = pl.multiple_of(base + i * K, 8)
        pltpu.sync_copy(vidx_hbm.at[pl.ds(off, K)], idx_v)
        pltpu.sync_copy(eidx_hbm.at[pl.ds(off, K)], idx_e)
        pltpu.async_copy(table_hbm.at[idx_v], rows_v, sem).wait()
        pltpu.sync_copy(rows_v, acc_sh.at[idx_e], add=True)
        return carry

    lax.fori_loop(0, CHUNKS, chunk, 0)
    plsc.subcore_barrier()

    def ocopy(j, carry):
        r0 = sid * ROWS_PER_TILE + j * OB
        pltpu.sync_copy(acc_sh.at[pl.ds(r0, OB)], stage_v)
        pltpu.sync_copy(stage_v, out_hbm.at[cid, pl.ds(r0, OB)])
        return carry

    lax.fori_loop(0, ROWS_PER_TILE // OB, ocopy, 0)


def _sc_aggregate(table, v_idx, e_idx):
    mesh = plsc.VectorSubcoreMesh(core_axis_name="c", subcore_axis_name="s")
    k = functools.partial(
        pl.kernel,
        mesh=mesh,
        out_type=jax.ShapeDtypeStruct((NC, EP, W), jnp.float32),
        scratch_types=[
            pltpu.VMEM((K,), jnp.int32),
            pltpu.VMEM((K,), jnp.int32),
            pltpu.VMEM((K, W), jnp.float32),
            pltpu.VMEM((OB, W), jnp.float32),
            pltpu.VMEM_SHARED((EP, W), jnp.float32),
            pltpu.SemaphoreType.DMA,
        ],
    )(_sc_body)
    return k(table, v_idx, e_idx)


R3 = 200
G3 = E // R3


def _loss_body(parts_ref, delta_ref, u_ref, o_ref, acc):
    i = pl.program_id(0)

    @pl.when(i == 0)
    def _():
        acc[0] = 0.0
        acc[1] = 0.0

    x = parts_ref[0] + parts_ref[1]
    counts = jnp.sum(x[:, 2 * C:], axis=1, keepdims=True)
    cnt = jnp.maximum(counts, 1.0)
    mean_s = x[:, :C] / cnt
    mean_t = x[:, C:2 * C] / cnt
    log_inp = jnp.log(mean_s / TAU + 1e-09)
    tgt = mean_t / TAU
    per_edge = jnp.sum(tgt * (jnp.log(tgt) - log_inp), axis=1, keepdims=True)

    p = jnp.clip(delta_ref[...], 0.0, 1.0)
    maskf = (u_ref[...] < p).astype(jnp.float32)
    acc[0] += jnp.sum(maskf * per_edge)
    acc[1] += jnp.sum(maskf)

    @pl.when(i == G3 - 1)
    def _():
        n = acc[1]
        loss = acc[0] / jnp.maximum(n, 1.0)
        o_ref[0, 0] = jnp.where(n > 0.0, loss, 0.0)


def _finalize(parts, delta_col, u_col):
    out = pl.pallas_call(
        _loss_body,
        grid=(G3,),
        in_specs=[pl.BlockSpec((NC, R3, W), lambda i: (0, i, 0)),
                  pl.BlockSpec((R3, 1), lambda i: (i, 0)),
                  pl.BlockSpec((R3, 1), lambda i: (i, 0))],
        out_specs=pl.BlockSpec((1, 1), lambda i: (0, 0)),
        out_shape=jax.ShapeDtypeStruct((1, 1), jnp.float32),
        scratch_shapes=[pltpu.SMEM((2,), jnp.float32)],
    )(parts, delta_col, u_col)
    return out[0, 0]


def kernel(pred_s, pred_t, delta_e_, v_idx, e_idx):
    table = _build_table(pred_s, pred_t)
    parts = _sc_aggregate(table, v_idx, e_idx)
    u = jax.random.uniform(jax.random.key(42), (E,), jnp.float32)
    return _finalize(parts, delta_e_[:, None], u[:, None])


# speedup vs baseline: 7.5220x; 7.5220x over previous
import functools

import jax
import jax.numpy as jnp
from jax import lax
from jax.experimental import pallas as pl
from jax.experimental.pallas import tpu as pltpu
from jax.experimental.pallas import tpu_sc as plsc

N = 10000
C = 128
P = 320000
E = 5000
TAU = 1.0

NC = 2
NS = 16
NW = NC * NS
K = 80
PER_W = P // NW
CHUNKS = PER_W // K
W = 2 * C
EP = 5120
ROWS_PER_TILE = EP // NS
OB = 32
L = 16
HB = EP // 128


def _table_body(s_ref, t_ref, o_ref):
    def softmax(x):
        m = jnp.max(x, axis=1, keepdims=True)
        ex = jnp.exp(x - m)
        return ex / jnp.sum(ex, axis=1, keepdims=True)

    o_ref[...] = jnp.concatenate([softmax(s_ref[...]), softmax(t_ref[...])],
                                 axis=1)


def _build_table(pred_s, pred_t):
    R = 400
    return pl.pallas_call(
        _table_body,
        grid=(N // R,),
        in_specs=[pl.BlockSpec((R, C), lambda i: (i, 0)),
                  pl.BlockSpec((R, C), lambda i: (i, 0))],
        out_specs=pl.BlockSpec((R, W), lambda i: (i, 0)),
        out_shape=jax.ShapeDtypeStruct((N, W), jnp.float32),
    )(pred_s, pred_t)


def _sc_body(table_hbm, vidx_hbm, eidx_hbm, out_hbm,
             idx_v, idx_e, rows_v, stage_v, acc_sh, sem):
    cid = lax.axis_index("c")
    sid = lax.axis_index("s")
    wid = sid * NC + cid

    zero = jnp.zeros((L,), jnp.float32)

    def zstore(i, carry):
        r = i // (W // L)
        c = i % (W // L)
        stage_v[r, pl.ds(c * L, L)] = zero
        return carry

    lax.fori_loop(0, OB * (W // L), zstore, 0)

    def zcopy(j, carry):
        r0 = sid * ROWS_PER_TILE + j * OB
        pltpu.sync_copy(stage_v, acc_sh.at[pl.ds(r0, OB)])
        return carry

    lax.fori_loop(0, ROWS_PER_TILE // OB, zcopy, 0)
    plsc.subcore_barrier()

    base = wid * PER_W

    def chunk(i, carry):
        off = pl.multiple_of(base + i * K, 8)
        pltpu.sync_copy(vidx_hbm.at[pl.ds(off, K)], idx_v)
        pltpu.sync_copy(eidx_hbm.at[pl.ds(off, K)], idx_e)
        pltpu.async_copy(table_hbm.at[idx_v], rows_v, sem).wait()
        pltpu.sync_copy(rows_v, acc_sh.at[idx_e], add=True)
        return carry

    lax.fori_loop(0, CHUNKS, chunk, 0)
    plsc.subcore_barrier()

    def ocopy(j, carry):
        r0 = sid * ROWS_PER_TILE + j * OB
        pltpu.sync_copy(acc_sh.at[pl.ds(r0, OB)], stage_v)
        pltpu.sync_copy(stage_v, out_hbm.at[cid, pl.ds(r0, OB)])
        return carry

    lax.fori_loop(0, ROWS_PER_TILE // OB, ocopy, 0)


def _sc_aggregate(table, v_idx, e_idx):
    mesh = plsc.VectorSubcoreMesh(core_axis_name="c", subcore_axis_name="s")
    k = functools.partial(
        pl.kernel,
        mesh=mesh,
        compiler_params=pltpu.CompilerParams(use_tc_tiling_on_sc=False),
        out_type=jax.ShapeDtypeStruct((NC, EP, W), jnp.float32),
        scratch_types=[
            pltpu.VMEM((K,), jnp.int32),
            pltpu.VMEM((K,), jnp.int32),
            pltpu.VMEM((K, W), jnp.float32),
            pltpu.VMEM((OB, W), jnp.float32),
            pltpu.VMEM_SHARED((EP, W), jnp.float32),
            pltpu.SemaphoreType.DMA,
        ],
    )(_sc_body)
    return k(table, v_idx, e_idx)


PB = 2560
GC = P // PB


def _counts_body(e_ref, o_ref):
    i = pl.program_id(0)

    @pl.when(i == 0)
    def _():
        o_ref[...] = jnp.zeros((HB, C), jnp.float32)

    x = e_ref[0]
    hi = x // C
    lo = x - hi * C
    oh_hi = (jnp.broadcast_to(hi, (HB, PB))
             == lax.broadcasted_iota(jnp.int32, (HB, PB), 0)
             ).astype(jnp.float32)
    oh_lo = (jnp.broadcast_to(lo, (C, PB))
             == lax.broadcasted_iota(jnp.int32, (C, PB), 0)
             ).astype(jnp.float32)
    o_ref[...] += lax.dot_general(oh_hi, oh_lo, (((1,), (1,)), ((), ())),
                                  preferred_element_type=jnp.float32)


def _edge_counts(e_idx):
    e3 = e_idx.reshape(GC, 1, PB)
    return pl.pallas_call(
        _counts_body,
        grid=(GC,),
        in_specs=[pl.BlockSpec((1, 1, PB), lambda i: (i, 0, 0))],
        out_specs=pl.BlockSpec((HB, C), lambda i: (0, 0)),
        out_shape=jax.ShapeDtypeStruct((HB, C), jnp.float32),
    )(e3)


R3 = 200
G3 = E // R3


def _loss_body(parts_ref, cnt_ref, delta_ref, u_ref, o_ref, acc):
    i = pl.program_id(0)

    @pl.when(i == 0)
    def _():
        acc[0] = 0.0
        acc[1] = 0.0

    x = parts_ref[0] + parts_ref[1]
    cnt = jnp.maximum(cnt_ref[...], 1.0)
    mean_s = x[:, :C] / cnt
    mean_t = x[:, C:] / cnt
    log_inp = jnp.log(mean_s / TAU + 1e-09)
    tgt = mean_t / TAU
    per_edge = jnp.sum(tgt * (jnp.log(tgt) - log_inp), axis=1, keepdims=True)

    p = jnp.clip(delta_ref[...], 0.0, 1.0)
    maskf = (u_ref[...] < p).astype(jnp.float32)
    acc[0] += jnp.sum(maskf * per_edge)
    acc[1] += jnp.sum(maskf)

    @pl.when(i == G3 - 1)
    def _():
        n = acc[1]
        loss = acc[0] / jnp.maximum(n, 1.0)
        o_ref[...] = jnp.full((1, 1), jnp.where(n > 0.0, loss, 0.0),
                              dtype=jnp.float32)


def _finalize(parts, cnt_col, delta_col, u_col):
    out = pl.pallas_call(
        _loss_body,
        grid=(G3,),
        in_specs=[pl.BlockSpec((NC, R3, W), lambda i: (0, i, 0)),
                  pl.BlockSpec((R3, 1), lambda i: (i, 0)),
                  pl.BlockSpec((R3, 1), lambda i: (i, 0)),
                  pl.BlockSpec((R3, 1), lambda i: (i, 0))],
        out_specs=pl.BlockSpec((1, 1), lambda i: (0, 0)),
        out_shape=jax.ShapeDtypeStruct((1, 1), jnp.float32),
        scratch_shapes=[pltpu.SMEM((2,), jnp.float32)],
    )(parts, cnt_col, delta_col, u_col)
    return out[0, 0]


def kernel(pred_s, pred_t, delta_e_, v_idx, e_idx):
    table = _build_table(pred_s, pred_t)
    parts = _sc_aggregate(table, v_idx, e_idx)
    counts = _edge_counts(e_idx).reshape(EP, 1)[:E]
    u = jax.random.uniform(jax.random.key(42), (E,), jnp.float32)
    return _finalize(parts, counts, delta_e_[:, None], u[:, None])


# speedup vs baseline: 9.8617x; 1.3110x over previous
import functools

import jax
import jax.numpy as jnp
from jax import lax
from jax.experimental import pallas as pl
from jax.experimental.pallas import tpu as pltpu
from jax.experimental.pallas import tpu_sc as plsc

N = 10000
C = 128
P = 320000
E = 5000
TAU = 1.0

NC = 2
NS = 16
NW = NC * NS
K = 40
PER_W = P // NW
CHUNKS = PER_W // K
W = 2 * C
EP = 5120
ROWS_PER_TILE = EP // NS
OB = 32
L = 16
HB = EP // 128


def _table_body(s_ref, t_ref, o_ref):
    def softmax(x):
        m = jnp.max(x, axis=1, keepdims=True)
        ex = jnp.exp(x - m)
        return ex / jnp.sum(ex, axis=1, keepdims=True)

    o_ref[...] = jnp.concatenate([softmax(s_ref[...]), softmax(t_ref[...])],
                                 axis=1)


def _build_table(pred_s, pred_t):
    R = 400
    return pl.pallas_call(
        _table_body,
        grid=(N // R,),
        in_specs=[pl.BlockSpec((R, C), lambda i: (i, 0)),
                  pl.BlockSpec((R, C), lambda i: (i, 0))],
        out_specs=pl.BlockSpec((R, W), lambda i: (i, 0)),
        out_shape=jax.ShapeDtypeStruct((N, W), jnp.float32),
    )(pred_s, pred_t)


def _sc_body(table_hbm, vidx_hbm, eidx_hbm, out_hbm,
             vidx_b, eidx_b, rows0, rows1, acc_sh,
             sem_i, sem_g0, sem_g1, sem_s0, sem_s1):
    cid = lax.axis_index("c")
    sid = lax.axis_index("s")
    wid = sid * NC + cid

    idx_load = pltpu.async_copy(vidx_hbm.at[wid], vidx_b, sem_i)
    idx_load2 = pltpu.async_copy(eidx_hbm.at[wid], eidx_b, sem_i)

    zero = jnp.zeros((L,), jnp.float32)
    stage_v = rows0.at[pl.ds(0, OB)]

    def zstore(i, carry):
        r = i // (W // L)
        c = i % (W // L)
        rows0[r, pl.ds(c * L, L)] = zero
        return carry

    lax.fori_loop(0, OB * (W // L), zstore, 0)

    def zcopy(j, carry):
        r0 = sid * ROWS_PER_TILE + j * OB
        pltpu.sync_copy(stage_v, acc_sh.at[pl.ds(r0, OB)])
        return carry

    lax.fori_loop(0, ROWS_PER_TILE // OB, zcopy, 0)
    idx_load.wait()
    idx_load2.wait()
    plsc.subcore_barrier()

    NP = CHUNKS // 2

    pltpu.async_copy(table_hbm.at[vidx_b.at[0]], rows0, sem_g0)

    def pair(j, carry):
        c1 = 2 * j + 1
        pltpu.make_async_copy(table_hbm.at[vidx_b.at[0]], rows0, sem_g0).wait()

        @pl.when(j > 0)
        def _():
            pltpu.make_async_copy(rows1, acc_sh.at[eidx_b.at[0]], sem_s1).wait()

        pltpu.async_copy(table_hbm.at[vidx_b.at[c1]], rows1, sem_g1)
        pltpu.async_copy(rows0, acc_sh.at[eidx_b.at[2 * j]], sem_s0, add=True)
        pltpu.make_async_copy(table_hbm.at[vidx_b.at[0]], rows1, sem_g1).wait()

        @pl.when(j + 1 < NP)
        def _():
            pltpu.make_async_copy(rows0, acc_sh.at[eidx_b.at[0]], sem_s0).wait()
            pltpu.async_copy(table_hbm.at[vidx_b.at[2 * j + 2]], rows0, sem_g0)

        pltpu.async_copy(rows1, acc_sh.at[eidx_b.at[c1]], sem_s1, add=True)
        return carry

    lax.fori_loop(0, NP, pair, 0)
    pltpu.make_async_copy(rows0, acc_sh.at[eidx_b.at[0]], sem_s0).wait()
    pltpu.make_async_copy(rows1, acc_sh.at[eidx_b.at[0]], sem_s1).wait()
    plsc.subcore_barrier()

    def ocopy(j, carry):
        r0 = sid * ROWS_PER_TILE + j * OB
        pltpu.sync_copy(acc_sh.at[pl.ds(r0, OB)], stage_v)
        pltpu.sync_copy(stage_v, out_hbm.at[cid, pl.ds(r0, OB)])
        return carry

    lax.fori_loop(0, ROWS_PER_TILE // OB, ocopy, 0)


def _sc_aggregate(table, v_idx, e_idx):
    mesh = plsc.VectorSubcoreMesh(core_axis_name="c", subcore_axis_name="s")
    k = functools.partial(
        pl.kernel,
        mesh=mesh,
        compiler_params=pltpu.CompilerParams(use_tc_tiling_on_sc=False),
        out_type=jax.ShapeDtypeStruct((NC, EP, W), jnp.float32),
        scratch_types=[
            pltpu.VMEM((CHUNKS, K), jnp.int32),
            pltpu.VMEM((CHUNKS, K), jnp.int32),
            pltpu.VMEM((K, W), jnp.float32),
            pltpu.VMEM((K, W), jnp.float32),
            pltpu.VMEM_SHARED((EP, W), jnp.float32),
            pltpu.SemaphoreType.DMA,
            pltpu.SemaphoreType.DMA,
            pltpu.SemaphoreType.DMA,
            pltpu.SemaphoreType.DMA,
            pltpu.SemaphoreType.DMA,
        ],
    )(_sc_body)
    return k(table, v_idx.reshape(NW, CHUNKS, K), e_idx.reshape(NW, CHUNKS, K))


PB = 2560
GC = P // PB


def _counts_body(e_ref, o_ref):
    i = pl.program_id(0)

    @pl.when(i == 0)
    def _():
        o_ref[...] = jnp.zeros((HB, C), jnp.float32)

    x = e_ref[0]
    hi = x // C
    lo = x - hi * C
    oh_hi = (jnp.broadcast_to(hi, (HB, PB))
             == lax.broadcasted_iota(jnp.int32, (HB, PB), 0)
             ).astype(jnp.float32)
    oh_lo = (jnp.broadcast_to(lo, (C, PB))
             == lax.broadcasted_iota(jnp.int32, (C, PB), 0)
             ).astype(jnp.float32)
    o_ref[...] += lax.dot_general(oh_hi, oh_lo, (((1,), (1,)), ((), ())),
                                  preferred_element_type=jnp.float32)


def _edge_counts(e_idx):
    e3 = e_idx.reshape(GC, 1, PB)
    return pl.pallas_call(
        _counts_body,
        grid=(GC,),
        in_specs=[pl.BlockSpec((1, 1, PB), lambda i: (i, 0, 0))],
        out_specs=pl.BlockSpec((HB, C), lambda i: (0, 0)),
        out_shape=jax.ShapeDtypeStruct((HB, C), jnp.float32),
    )(e3)


R3 = 200
G3 = E // R3


def _loss_body(parts_ref, cnt_ref, delta_ref, u_ref, o_ref, acc):
    i = pl.program_id(0)

    @pl.when(i == 0)
    def _():
        acc[0] = 0.0
        acc[1] = 0.0

    x = parts_ref[0] + parts_ref[1]
    cnt = jnp.maximum(cnt_ref[...], 1.0)
    mean_s = x[:, :C] / cnt
    mean_t = x[:, C:] / cnt
    log_inp = jnp.log(mean_s / TAU + 1e-09)
    tgt = mean_t / TAU
    per_edge = jnp.sum(tgt * (jnp.log(tgt) - log_inp), axis=1, keepdims=True)

    p = jnp.clip(delta_ref[...], 0.0, 1.0)
    maskf = (u_ref[...] < p).astype(jnp.float32)
    acc[0] += jnp.sum(maskf * per_edge)
    acc[1] += jnp.sum(maskf)

    @pl.when(i == G3 - 1)
    def _():
        n = acc[1]
        loss = acc[0] / jnp.maximum(n, 1.0)
        o_ref[...] = jnp.full((1, 1), jnp.where(n > 0.0, loss, 0.0),
                              dtype=jnp.float32)


def _finalize(parts, cnt_col, delta_col, u_col):
    out = pl.pallas_call(
        _loss_body,
        grid=(G3,),
        in_specs=[pl.BlockSpec((NC, R3, W), lambda i: (0, i, 0)),
                  pl.BlockSpec((R3, 1), lambda i: (i, 0)),
                  pl.BlockSpec((R3, 1), lambda i: (i, 0)),
                  pl.BlockSpec((R3, 1), lambda i: (i, 0))],
        out_specs=pl.BlockSpec((1, 1), lambda i: (0, 0)),
        out_shape=jax.ShapeDtypeStruct((1, 1), jnp.float32),
        scratch_shapes=[pltpu.SMEM((2,), jnp.float32)],
    )(parts, cnt_col, delta_col, u_col)
    return out[0, 0]


def kernel(pred_s, pred_t, delta_e_, v_idx, e_idx):
    table = _build_table(pred_s, pred_t)
    parts = _sc_aggregate(table, v_idx, e_idx)
    counts = _edge_counts(e_idx).reshape(EP, 1)[:E]
    u = jax.random.uniform(jax.random.key(42), (E,), jnp.float32)
    return _finalize(parts, counts, delta_e_[:, None], u[:, None])
